# baseline (speedup 1.0000x reference)
import functools

import jax
import jax.numpy as jnp
from jax import lax
from jax.experimental import pallas as pl
from jax.experimental.pallas import tpu as pltpu
from jax.experimental.pallas import tpu_sc as plsc

N = 10000
D = 128
E = 320000
O = 40
OP = 48
HID = 128

NC = 2
NS = 16
NW = NC * NS
K = 128
CH = 80
CHH = CH // 2
EPAD = NW * CH * K
ABSORB = N
RS = 632
NPAD = NS * RS
DS = 640
NDPAD = NS * DS


def _sc_agg(dcols, with_deg):
    mesh = plsc.VectorSubcoreMesh(core_axis_name="c", subcore_axis_name="s")
    out_type = jax.ShapeDtypeStruct((NC, NPAD, dcols), jnp.float32)
    if with_deg:
        out_type = [out_type]
    scratch = [
        pltpu.VMEM((CHH, K), jnp.int32),
        pltpu.VMEM((CHH, K), jnp.int32),
        pltpu.VMEM((K, dcols), jnp.float32),
        pltpu.VMEM((K, dcols), jnp.float32),
        pltpu.VMEM_SHARED((NPAD, dcols), jnp.float32),
        pltpu.SemaphoreType.DMA,
        pltpu.SemaphoreType.DMA,
    ]
    if with_deg:
        out_type.append(jax.ShapeDtypeStruct((NC, NDPAD), jnp.float32))
        scratch += [
            pltpu.VMEM((K,), jnp.float32),
            pltpu.VMEM((DS,), jnp.float32),
            pltpu.VMEM_SHARED((NDPAD,), jnp.float32),
        ]

    @functools.partial(
        pl.kernel, out_type=out_type, mesh=mesh, scratch_types=scratch,
        compiler_params=pltpu.CompilerParams(
            use_tc_tiling_on_sc=(dcols % 128 == 0)))
    def agg(h_hbm, src_hbm, dst_hbm, zero_hbm, *refs):
        if with_deg:
            (out_hbm, deg_hbm, src_v, dst_v, rows_v, rows_w, accum,
             sem, sem2, ones_v, zv, dacc) = refs
        else:
            (out_hbm, src_v, dst_v, rows_v, rows_w, accum, sem, sem2) = refs
        c = lax.axis_index("c")
        s = lax.axis_index("s")
        wid = c * NS + s
        base = s * RS
        pltpu.async_copy(src_hbm.at[wid, pl.ds(0, CHH)], src_v, sem2)
        pltpu.async_copy(dst_hbm.at[wid, pl.ds(0, CHH)], dst_v, sem2)
        pltpu.sync_copy(zero_hbm, rows_v)
        for t in range(RS // K):
            pltpu.sync_copy(rows_v, accum.at[pl.ds(base + t * K, K)])
        rem = RS - (RS // K) * K
        pltpu.sync_copy(rows_v.at[pl.ds(0, rem)],
                        accum.at[pl.ds(base + (RS // K) * K, rem)])
        if with_deg:
            ones16 = jnp.ones((16,), jnp.float32)
            zero16 = jnp.zeros((16,), jnp.float32)

            def fill_ones(i, carry):
                ones_v[pl.ds(i * 16, 16)] = ones16
                return carry

            lax.fori_loop(0, K // 16, fill_ones, 0)

            def fill_zero(i, carry):
                zv[pl.ds(i * 16, 16)] = zero16
                return carry

            lax.fori_loop(0, DS // 16, fill_zero, 0)
            pltpu.sync_copy(zv, dacc.at[pl.ds(s * DS, DS)])
        plsc.subcore_barrier()

        for half in range(2):
            if half == 0:
                pltpu.make_async_copy(
                    src_hbm.at[wid, pl.ds(0, CHH)], src_v, sem2).wait()
                pltpu.make_async_copy(
                    dst_hbm.at[wid, pl.ds(0, CHH)], dst_v, sem2).wait()
            else:
                pltpu.sync_copy(src_hbm.at[wid, pl.ds(half * CHH, CHH)], src_v)
                pltpu.sync_copy(dst_hbm.at[wid, pl.ds(half * CHH, CHH)], dst_v)
            pltpu.async_copy(h_hbm.at[src_v.at[0]], rows_v, sem)

            def body(t, carry):
                g = 2 * t
                pltpu.async_copy(h_hbm.at[src_v.at[g + 1]], rows_w, sem2)
                pltpu.make_async_copy(h_hbm.at[src_v.at[g]], rows_v, sem).wait()
                pltpu.sync_copy(rows_v, accum.at[dst_v.at[g]], add=True)
                if with_deg:
                    pltpu.sync_copy(ones_v, dacc.at[dst_v.at[g]], add=True)

                @pl.when(g + 2 < CHH)
                def _():
                    pltpu.async_copy(h_hbm.at[src_v.at[g + 2]], rows_v, sem)

                pltpu.make_async_copy(h_hbm.at[src_v.at[g + 1]], rows_w, sem2).wait()
                pltpu.sync_copy(rows_w, accum.at[dst_v.at[g + 1]], add=True)
                if with_deg:
                    pltpu.sync_copy(ones_v, dacc.at[dst_v.at[g + 1]], add=True)
                return carry

            lax.fori_loop(0, CHH // 2, body, 0)
        plsc.subcore_barrier()
        pltpu.sync_copy(accum.at[pl.ds(base, RS)],
                        out_hbm.at[c, pl.ds(base, RS)])
        if with_deg:
            pltpu.sync_copy(dacc.at[pl.ds(s * DS, DS)],
                            deg_hbm.at[c, pl.ds(s * DS, DS)])

    return agg


RB = 2000
GRID = N // RB


def _log_softmax(y):
    z = y - jnp.max(y, axis=1, keepdims=True)
    return z - jnp.log(jnp.sum(jnp.exp(z), axis=1, keepdims=True))


def _tc_update1_body(x_ref, a0_ref, a1_ref, d0_ref, d1_ref, we0_ref, be0_ref,
                     we_ref, be_ref, wc_ref, bc_ref, out0_ref, out_ref, h_ref):
    y0 = jnp.dot(x_ref[...], we0_ref[...],
                 preferred_element_type=jnp.float32) + be0_ref[...]
    out0_ref[...] = _log_softmax(y0)
    deg = jnp.maximum(d0_ref[...] + d1_ref[...], 1.0)
    a = (a0_ref[0] + a1_ref[0]) / deg
    y = jnp.dot(a, we_ref[...], preferred_element_type=jnp.float32) + be_ref[...]
    out_ref[...] = _log_softmax(y)
    h = jnp.dot(a, wc_ref[...], preferred_element_type=jnp.float32) + bc_ref[...]
    h_ref[...] = jnp.maximum(h, 0.0)


def _tc_update2_body(a0_ref, a1_ref, d0_ref, d1_ref, we_ref, be_ref,
                     wc_ref, bc_ref, wep_ref, out_ref, h_ref, g_ref):
    deg = jnp.maximum(d0_ref[...] + d1_ref[...], 1.0)
    a = (a0_ref[0] + a1_ref[0]) / deg
    y = jnp.dot(a, we_ref[...], preferred_element_type=jnp.float32) + be_ref[...]
    out_ref[...] = _log_softmax(y)
    h = jnp.dot(a, wc_ref[...], preferred_element_type=jnp.float32) + bc_ref[...]
    h = jnp.maximum(h, 0.0)
    h_ref[...] = h
    g_ref[...] = jnp.dot(h, wep_ref[...], preferred_element_type=jnp.float32)


def _tc_exit3_body(g0_ref, g1_ref, d0_ref, d1_ref, be_ref,
                   o0_ref, o1_ref, o2_ref, out_ref):
    deg = jnp.maximum(d0_ref[...] + d1_ref[...], 1.0)
    a = (g0_ref[0, :, :O] + g1_ref[0, :, :O]) / deg
    out_ref[:, 0, :] = o0_ref[...]
    out_ref[:, 1, :] = o1_ref[...]
    out_ref[:, 2, :] = o2_ref[...]
    out_ref[:, 3, :] = _log_softmax(a + be_ref[...])


def _row_spec(shape):
    return pl.BlockSpec((RB,) + shape[1:], lambda i: (i,) + (0,) * (len(shape) - 1))


def _a_spec(core, dcols):
    return pl.BlockSpec((1, RB, dcols), lambda i, _c=core: (_c, i, 0))


_D_SPEC = pl.BlockSpec((RB, 1), lambda i: (i, 0))


def _w_spec(din, dout):
    return pl.BlockSpec((din, dout), lambda i: (0, 0))


def _tc_update1(x, a, d0, d1, we0, be0, we, be, wc, bc):
    return pl.pallas_call(
        _tc_update1_body,
        grid=(GRID,),
        in_specs=[_row_spec((N, D)), _a_spec(0, D), _a_spec(1, D),
                  _D_SPEC, _D_SPEC, _w_spec(D, O), _w_spec(1, O),
                  _w_spec(D, O), _w_spec(1, O), _w_spec(D, HID), _w_spec(1, HID)],
        out_specs=[_row_spec((N, O)), _row_spec((N, O)), _row_spec((N, HID))],
        out_shape=[jax.ShapeDtypeStruct((N, O), jnp.float32),
                   jax.ShapeDtypeStruct((N, O), jnp.float32),
                   jax.ShapeDtypeStruct((N, HID), jnp.float32)],
    )(x, a, a, d0, d1, we0, be0.reshape(1, O), we, be.reshape(1, O),
      wc, bc.reshape(1, HID))


def _tc_update2(a, d0, d1, we, be, wc, bc, wep):
    return pl.pallas_call(
        _tc_update2_body,
        grid=(GRID,),
        in_specs=[_a_spec(0, D), _a_spec(1, D), _D_SPEC, _D_SPEC,
                  _w_spec(D, O), _w_spec(1, O), _w_spec(D, HID), _w_spec(1, HID),
                  _w_spec(D, OP)],
        out_specs=[_row_spec((N, O)), _row_spec((N, HID)), _row_spec((N, OP))],
        out_shape=[jax.ShapeDtypeStruct((N, O), jnp.float32),
                   jax.ShapeDtypeStruct((N, HID), jnp.float32),
                   jax.ShapeDtypeStruct((N, OP), jnp.float32)],
    )(a, a, d0, d1, we, be.reshape(1, O), wc, bc.reshape(1, HID), wep)


def _tc_exit3(g, d0, d1, be, o0, o1, o2):
    return pl.pallas_call(
        _tc_exit3_body,
        grid=(GRID,),
        in_specs=[_a_spec(0, OP), _a_spec(1, OP), _D_SPEC, _D_SPEC,
                  _w_spec(1, O), _row_spec((N, O)), _row_spec((N, O)),
                  _row_spec((N, O))],
        out_specs=pl.BlockSpec((RB, 4, O), lambda i: (i, 0, 0)),
        out_shape=jax.ShapeDtypeStruct((N, 4, O), jnp.float32),
    )(g, g, d0, d1, be.reshape(1, O), o0, o1, o2)


def kernel(x, edge_index, We0, be0, We1, be1, We2, be2, We3, be3,
           Wc0, bc0, Wc1, bc1, Wc2, bc2):
    src = edge_index[0]
    dst = edge_index[1]
    pad = EPAD - E
    pad_src = jnp.arange(pad, dtype=jnp.int32) % N
    pad_dst = ABSORB + jnp.arange(pad, dtype=jnp.int32) % (NPAD - N)
    src3 = jnp.concatenate([src, pad_src]).reshape(NW, CH, K)
    dst3 = jnp.concatenate([dst, pad_dst]).reshape(NW, CH, K)
    zeros_kd = jnp.zeros((K, D), jnp.float32)
    zeros_kp = jnp.zeros((K, OP), jnp.float32)
    We3p = jnp.pad(We3, ((0, 0), (0, OP - O)))

    agg_deg = _sc_agg(D, True)
    agg_d = _sc_agg(D, False)
    agg_o = _sc_agg(OP, False)

    a1, degp = agg_deg(x, src3, dst3, zeros_kd)
    d0 = degp[0].reshape(NDPAD, 1)
    d1 = degp[1].reshape(NDPAD, 1)
    out0, out1, h1 = _tc_update1(x, a1, d0, d1, We0, be0, We1, be1, Wc0, bc0)
    a2 = agg_d(h1, src3, dst3, zeros_kd)
    out2, h2, g2 = _tc_update2(a2, d0, d1, We2, be2, Wc1, bc1, We3p)
    g3 = agg_o(g2, src3, dst3, zeros_kp)
    return _tc_exit3(g3, d0, d1, be3, out0, out1, out2)


# speedup vs baseline: 1.0455x; 1.0455x over previous
import functools

import jax
import jax.numpy as jnp
from jax import lax
from jax.experimental import pallas as pl
from jax.experimental.pallas import tpu as pltpu
from jax.experimental.pallas import tpu_sc as plsc

N = 10000
D = 128
E = 320000
O = 40
OP = 48
HID = 128

NC = 2
NS = 16
NW = NC * NS
K = 128
CH = 80
CHH = CH // 2
EPAD = NW * CH * K
ABSORB = N
RS = 632
NPAD = NS * RS
DS = 640
NDPAD = NS * DS


def _sc_agg(dcols, with_deg):
    mesh = plsc.VectorSubcoreMesh(core_axis_name="c", subcore_axis_name="s")
    out_type = jax.ShapeDtypeStruct((NC, NPAD, dcols), jnp.float32)
    if with_deg:
        out_type = [out_type]
    scratch = [
        pltpu.VMEM((CHH, K), jnp.int32),
        pltpu.VMEM((CHH, K), jnp.int32),
        pltpu.VMEM((K, dcols), jnp.float32),
        pltpu.VMEM((K, dcols), jnp.float32),
        pltpu.VMEM_SHARED((NPAD, dcols), jnp.float32),
        pltpu.SemaphoreType.DMA,
        pltpu.SemaphoreType.DMA,
    ]
    if with_deg:
        out_type.append(jax.ShapeDtypeStruct((NC, NDPAD), jnp.float32))
        scratch += [
            pltpu.VMEM((K,), jnp.float32),
            pltpu.VMEM((DS,), jnp.float32),
            pltpu.VMEM_SHARED((NDPAD,), jnp.float32),
        ]

    @functools.partial(
        pl.kernel, out_type=out_type, mesh=mesh, scratch_types=scratch,
        compiler_params=pltpu.CompilerParams(
            use_tc_tiling_on_sc=(dcols % 128 == 0)))
    def agg(h_hbm, src_hbm, dst_hbm, zero_hbm, *refs):
        if with_deg:
            (out_hbm, deg_hbm, src_v, dst_v, rows_v, rows_w, accum,
             sem, sem2, ones_v, zv, dacc) = refs
        else:
            (out_hbm, src_v, dst_v, rows_v, rows_w, accum, sem, sem2) = refs
        c = lax.axis_index("c")
        s = lax.axis_index("s")
        wid = c * NS + s
        base = s * RS
        pltpu.async_copy(src_hbm.at[wid, pl.ds(0, CHH)], src_v, sem2)
        pltpu.async_copy(dst_hbm.at[wid, pl.ds(0, CHH)], dst_v, sem2)
        pltpu.sync_copy(zero_hbm, rows_v)
        for t in range(RS // K):
            pltpu.sync_copy(rows_v, accum.at[pl.ds(base + t * K, K)])
        rem = RS - (RS // K) * K
        pltpu.sync_copy(rows_v.at[pl.ds(0, rem)],
                        accum.at[pl.ds(base + (RS // K) * K, rem)])
        if with_deg:
            ones16 = jnp.ones((16,), jnp.float32)
            zero16 = jnp.zeros((16,), jnp.float32)

            def fill_ones(i, carry):
                ones_v[pl.ds(i * 16, 16)] = ones16
                return carry

            lax.fori_loop(0, K // 16, fill_ones, 0)

            def fill_zero(i, carry):
                zv[pl.ds(i * 16, 16)] = zero16
                return carry

            lax.fori_loop(0, DS // 16, fill_zero, 0)
            pltpu.sync_copy(zv, dacc.at[pl.ds(s * DS, DS)])
        plsc.subcore_barrier()

        for half in range(2):
            if half == 0:
                pltpu.make_async_copy(
                    src_hbm.at[wid, pl.ds(0, CHH)], src_v, sem2).wait()
                pltpu.make_async_copy(
                    dst_hbm.at[wid, pl.ds(0, CHH)], dst_v, sem2).wait()
            else:
                pltpu.sync_copy(src_hbm.at[wid, pl.ds(half * CHH, CHH)], src_v)
                pltpu.sync_copy(dst_hbm.at[wid, pl.ds(half * CHH, CHH)], dst_v)
            pltpu.async_copy(h_hbm.at[src_v.at[0]], rows_v, sem)

            def body(t, carry):
                g = 2 * t
                pltpu.async_copy(h_hbm.at[src_v.at[g + 1]], rows_w, sem2)
                pltpu.make_async_copy(h_hbm.at[src_v.at[g]], rows_v, sem).wait()
                pltpu.sync_copy(rows_v, accum.at[dst_v.at[g]], add=True)
                if with_deg:
                    pltpu.sync_copy(ones_v, dacc.at[dst_v.at[g]], add=True)

                @pl.when(g + 2 < CHH)
                def _():
                    pltpu.async_copy(h_hbm.at[src_v.at[g + 2]], rows_v, sem)

                pltpu.make_async_copy(h_hbm.at[src_v.at[g + 1]], rows_w, sem2).wait()
                pltpu.sync_copy(rows_w, accum.at[dst_v.at[g + 1]], add=True)
                if with_deg:
                    pltpu.sync_copy(ones_v, dacc.at[dst_v.at[g + 1]], add=True)
                return carry

            lax.fori_loop(0, CHH // 2, body, 0)
        plsc.subcore_barrier()
        pltpu.sync_copy(accum.at[pl.ds(base, RS)],
                        out_hbm.at[c, pl.ds(base, RS)])
        if with_deg:
            pltpu.sync_copy(dacc.at[pl.ds(s * DS, DS)],
                            deg_hbm.at[c, pl.ds(s * DS, DS)])

    return agg


RB = 2000
GRID = N // RB


def _log_softmax(y):
    z = y - jnp.max(y, axis=1, keepdims=True)
    return z - jnp.log(jnp.sum(jnp.exp(z), axis=1, keepdims=True))


def _tc_update1_body(x_ref, a0_ref, a1_ref, d0_ref, d1_ref, we0_ref, be0_ref,
                     we_ref, be_ref, wc_ref, bc_ref, out0_ref, out_ref, h_ref):
    y0 = jnp.dot(x_ref[...], we0_ref[...],
                 preferred_element_type=jnp.float32) + be0_ref[...]
    out0_ref[...] = _log_softmax(y0)
    deg = jnp.maximum(d0_ref[...] + d1_ref[...], 1.0)
    a = (a0_ref[0] + a1_ref[0]) / deg
    y = jnp.dot(a, we_ref[...], preferred_element_type=jnp.float32) + be_ref[...]
    out_ref[...] = _log_softmax(y)
    h = jnp.dot(a, wc_ref[...], preferred_element_type=jnp.float32) + bc_ref[...]
    h_ref[...] = jnp.maximum(h, 0.0)


def _tc_update2_body(a0_ref, a1_ref, d0_ref, d1_ref, we_ref, be_ref,
                     wc_ref, bc_ref, wep_ref, out_ref, h_ref, g_ref):
    deg = jnp.maximum(d0_ref[...] + d1_ref[...], 1.0)
    a = (a0_ref[0] + a1_ref[0]) / deg
    y = jnp.dot(a, we_ref[...], preferred_element_type=jnp.float32) + be_ref[...]
    out_ref[...] = _log_softmax(y)
    h = jnp.dot(a, wc_ref[...], preferred_element_type=jnp.float32) + bc_ref[...]
    h = jnp.maximum(h, 0.0)
    h_ref[...] = h
    g_ref[...] = jnp.dot(h, wep_ref[...], preferred_element_type=jnp.float32)


def _tc_exit3_body(g0_ref, g1_ref, d0_ref, d1_ref, be_ref, out_ref):
    deg = jnp.maximum(d0_ref[...] + d1_ref[...], 1.0)
    a = (g0_ref[0, :, :O] + g1_ref[0, :, :O]) / deg
    out_ref[...] = _log_softmax(a + be_ref[...])


def _row_spec(shape):
    return pl.BlockSpec((RB,) + shape[1:], lambda i: (i,) + (0,) * (len(shape) - 1))


def _a_spec(core, dcols):
    return pl.BlockSpec((1, RB, dcols), lambda i, _c=core: (_c, i, 0))


_D_SPEC = pl.BlockSpec((RB, 1), lambda i: (i, 0))


def _w_spec(din, dout):
    return pl.BlockSpec((din, dout), lambda i: (0, 0))


def _tc_update1(x, a, d0, d1, we0, be0, we, be, wc, bc):
    return pl.pallas_call(
        _tc_update1_body,
        grid=(GRID,),
        in_specs=[_row_spec((N, D)), _a_spec(0, D), _a_spec(1, D),
                  _D_SPEC, _D_SPEC, _w_spec(D, O), _w_spec(1, O),
                  _w_spec(D, O), _w_spec(1, O), _w_spec(D, HID), _w_spec(1, HID)],
        out_specs=[_row_spec((N, O)), _row_spec((N, O)), _row_spec((N, HID))],
        out_shape=[jax.ShapeDtypeStruct((N, O), jnp.float32),
                   jax.ShapeDtypeStruct((N, O), jnp.float32),
                   jax.ShapeDtypeStruct((N, HID), jnp.float32)],
    )(x, a, a, d0, d1, we0, be0.reshape(1, O), we, be.reshape(1, O),
      wc, bc.reshape(1, HID))


def _tc_update2(a, d0, d1, we, be, wc, bc, wep):
    return pl.pallas_call(
        _tc_update2_body,
        grid=(GRID,),
        in_specs=[_a_spec(0, D), _a_spec(1, D), _D_SPEC, _D_SPEC,
                  _w_spec(D, O), _w_spec(1, O), _w_spec(D, HID), _w_spec(1, HID),
                  _w_spec(D, OP)],
        out_specs=[_row_spec((N, O)), _row_spec((N, HID)), _row_spec((N, OP))],
        out_shape=[jax.ShapeDtypeStruct((N, O), jnp.float32),
                   jax.ShapeDtypeStruct((N, HID), jnp.float32),
                   jax.ShapeDtypeStruct((N, OP), jnp.float32)],
    )(a, a, d0, d1, we, be.reshape(1, O), wc, bc.reshape(1, HID), wep)


def _tc_exit3(g, d0, d1, be):
    return pl.pallas_call(
        _tc_exit3_body,
        grid=(GRID,),
        in_specs=[_a_spec(0, OP), _a_spec(1, OP), _D_SPEC, _D_SPEC,
                  _w_spec(1, O)],
        out_specs=_row_spec((N, O)),
        out_shape=jax.ShapeDtypeStruct((N, O), jnp.float32),
    )(g, g, d0, d1, be.reshape(1, O))


def kernel(x, edge_index, We0, be0, We1, be1, We2, be2, We3, be3,
           Wc0, bc0, Wc1, bc1, Wc2, bc2):
    src = edge_index[0]
    dst = edge_index[1]
    pad = EPAD - E
    pad_src = jnp.arange(pad, dtype=jnp.int32) % N
    pad_dst = ABSORB + jnp.arange(pad, dtype=jnp.int32) % (NPAD - N)
    src3 = jnp.concatenate([src, pad_src]).reshape(NW, CH, K)
    dst3 = jnp.concatenate([dst, pad_dst]).reshape(NW, CH, K)
    zeros_kd = jnp.zeros((K, D), jnp.float32)
    zeros_kp = jnp.zeros((K, OP), jnp.float32)
    We3p = jnp.pad(We3, ((0, 0), (0, OP - O)))

    agg_deg = _sc_agg(D, True)
    agg_d = _sc_agg(D, False)
    agg_o = _sc_agg(OP, False)

    a1, degp = agg_deg(x, src3, dst3, zeros_kd)
    d0 = degp[0].reshape(NDPAD, 1)
    d1 = degp[1].reshape(NDPAD, 1)
    out0, out1, h1 = _tc_update1(x, a1, d0, d1, We0, be0, We1, be1, Wc0, bc0)
    a2 = agg_d(h1, src3, dst3, zeros_kd)
    out2, h2, g2 = _tc_update2(a2, d0, d1, We2, be2, Wc1, bc1, We3p)
    g3 = agg_o(g2, src3, dst3, zeros_kp)
    out3 = _tc_exit3(g3, d0, d1, be3)
    return jnp.stack([out0, out1, out2, out3], axis=1)
